# baseline (speedup 1.0000x reference)
import functools

import jax
import jax.numpy as jnp
from jax.experimental import pallas as pl
from jax.experimental.pallas import tpu as pltpu

N_NODES = 10
D_FEAT = 131

TN1, G1 = 256, 8
TN2, G2 = 256, 10
TN3, G3 = 256, 20
TN4, G4 = 128, 20
P1, P2, P3 = G1, G1 + G2, G1 + G2 + G3
STEPS = G1 + G2 + G3 + G4


def _gemv(x, w):
    return jax.lax.dot_general(
        x, w, (((1,), (1,)), ((), ())), preferred_element_type=jnp.float32)


def _gather_kernel(smat_ref, idx_ref, out_ref):
    iota = jax.lax.broadcasted_iota(jnp.int32, (16, 16), 1)
    acc = jnp.zeros((16, 16), jnp.float32)
    for t in range(4):
        acc = acc + (idx_ref[:, t:t + 1] == iota).astype(jnp.float32)
    out_ref[...] = jax.lax.dot_general(
        acc, smat_ref[...], (((1,), (0,)), ((), ())),
        preferred_element_type=jnp.float32) * 0.25


def _mesh1_kernel(a1_ref, f_ref, w1_ref, w2_ref, w3_ref, w4_ref,
                  b1_ref, b2_ref, b3_ref, b4_ref,
                  out1_ref, out2_ref, h1, h2):
    s = pl.program_id(0)

    @pl.when(s < P1)
    def _phase1():
        h1[:, pl.ds(s * TN1, TN1)] = jax.nn.relu(
            _gemv(a1_ref[...], w1_ref[...]) + b1_ref[...])

    @pl.when((s >= P1) & (s < P2))
    def _phase2():
        out1_ref[...] = _gemv(h1[:, :2000], w2_ref[...]) + b2_ref[...]

    @pl.when((s >= P2) & (s < P3))
    def _phase3():
        h2[:, pl.ds((s - P2) * TN3, TN3)] = jax.nn.relu(
            _gemv(f_ref[...], w3_ref[...]) + b3_ref[...])

    @pl.when(s >= P3)
    def _phase4():
        out2_ref[...] = _gemv(h2[...], w4_ref[...]) + b4_ref[...]


@functools.partial(jax.jit, static_argnames=("interpret",))
def _run(spatial, structural, neighbour, W1, b1, W2, b2, W3, b3, W4, b4,
         interpret=False):
    a1 = jnp.concatenate([spatial, structural])[None, :]
    smat = jnp.zeros((16, D_FEAT), jnp.float32).at[:N_NODES].set(
        structural.reshape(N_NODES, D_FEAT))
    nbr = neighbour.reshape(N_NODES, 3)
    idx = jnp.concatenate(
        [jnp.arange(N_NODES, dtype=jnp.int32)[:, None], nbr], axis=1)
    idxp = jnp.full((16, 8), -1, jnp.int32).at[:N_NODES, :4].set(idx)

    f2d = pl.pallas_call(
        _gather_kernel,
        out_shape=jax.ShapeDtypeStruct((16, D_FEAT), jnp.float32),
        interpret=interpret,
    )(smat, idxp)
    f = f2d[:N_NODES].reshape(1, N_NODES * D_FEAT)

    const = lambda bs: pl.BlockSpec(bs, lambda s: (0, 0))
    out1, out2 = pl.pallas_call(
        _mesh1_kernel,
        grid=(STEPS,),
        in_specs=[
            const((1, 1950)),
            const((1, 1310)),
            pl.BlockSpec((TN1, 1950), lambda s: (jnp.minimum(s, G1 - 1), 0)),
            pl.BlockSpec((TN2, 2000), lambda s: (jnp.clip(s - P1, 0, G2 - 1), 0)),
            pl.BlockSpec((TN3, 1310), lambda s: (jnp.clip(s - P2, 0, G3 - 1), 0)),
            pl.BlockSpec((TN4, 5120), lambda s: (jnp.clip(s - P3, 0, G4 - 1), 0)),
            pl.BlockSpec((1, TN1), lambda s: (0, jnp.minimum(s, G1 - 1))),
            pl.BlockSpec((1, TN2), lambda s: (0, jnp.clip(s - P1, 0, G2 - 1))),
            pl.BlockSpec((1, TN3), lambda s: (0, jnp.clip(s - P2, 0, G3 - 1))),
            pl.BlockSpec((1, TN4), lambda s: (0, jnp.clip(s - P3, 0, G4 - 1))),
        ],
        out_specs=[
            pl.BlockSpec((1, TN2), lambda s: (0, jnp.clip(s - P1, 0, G2 - 1))),
            pl.BlockSpec((1, TN4), lambda s: (0, jnp.clip(s - P3, 0, G4 - 1))),
        ],
        out_shape=[
            jax.ShapeDtypeStruct((1, 2560), jnp.float32),
            jax.ShapeDtypeStruct((1, 2560), jnp.float32),
        ],
        scratch_shapes=[
            pltpu.VMEM((1, TN1 * G1), jnp.float32),
            pltpu.VMEM((1, 5120), jnp.float32),
        ],
        interpret=interpret,
    )(a1, f, W1, W2, W3, W4,
      b1[None, :], b2[None, :], b3[None, :], b4[None, :])
    return out1[0], out2[0]


def kernel(spatial, structural, neighbour, W1, b1, W2, b2, W3, b3, W4, b4):
    return _run(spatial, structural, neighbour,
                W1, b1, W2, b2, W3, b3, W4, b4)
